# Initial kernel scaffold; baseline (speedup 1.0000x reference)
import jax
import jax.numpy as jnp
from jax.experimental import pallas as pl


def kernel(x, user_emb, product_emb, model_emb, gender_emb, age_emb, residence_emb, color_emb, size_emb, material_emb, W1, b1, W2, b2):
    raise NotImplementedError("write your pallas kernel here")


# speedup vs baseline: 1.2890x; 1.2890x over previous
import functools

import jax
import jax.numpy as jnp
from jax import lax
from jax.experimental import pallas as pl
from jax.experimental.pallas import tpu as pltpu
from jax.experimental.pallas import tpu_sc as plsc

_B = 16384
_DIMS = (8, 8, 8, 2, 4, 3, 8, 4, 4)
_NTAB = len(_DIMS)
_NC, _NS = 2, 16
_NW = _NC * _NS
_BPW = _B // _NW
_ICH = 128
_NCHUNK = _BPW // _ICH


def _sc_gather(idx_flat, tables):
    mesh = plsc.VectorSubcoreMesh(
        core_axis_name="c", subcore_axis_name="s",
        num_cores=_NC, num_subcores=_NS)

    out_type = tuple(
        jax.ShapeDtypeStruct((_B, d), jnp.float32) for d in _DIMS)
    scratch = (
        [pltpu.VMEM((_NTAB * _BPW,), jnp.int32)]
        + [pltpu.VMEM((_BPW, d), jnp.float32) for d in _DIMS]
        + [pltpu.SemaphoreType.DMA, pltpu.SemaphoreType.DMA]
    )

    @functools.partial(
        pl.kernel, mesh=mesh, out_type=out_type, scratch_types=scratch,
        compiler_params=pltpu.CompilerParams(use_tc_tiling_on_sc=False))
    def gather_kernel(idx_hbm, *rest):
        tabs = rest[:_NTAB]
        outs = rest[_NTAB:2 * _NTAB]
        idx_v = rest[2 * _NTAB]
        rows = rest[2 * _NTAB + 1:3 * _NTAB + 1]
        sem = rest[3 * _NTAB + 1]
        sem2 = rest[3 * _NTAB + 2]

        wid = lax.axis_index("s") * _NC + lax.axis_index("c")
        base = wid * _BPW
        idx_copies = [
            pltpu.async_copy(
                idx_hbm.at[pl.ds(t * _B + base, _BPW)],
                idx_v.at[pl.ds(t * _BPW, _BPW)],
                sem2)
            for t in range(_NTAB)
        ]
        for c in idx_copies:
            c.wait()
        copies = []
        for t in range(_NTAB):
            for k in range(_NCHUNK):
                copies.append(pltpu.async_copy(
                    tabs[t].at[idx_v.at[pl.ds(t * _BPW + k * _ICH, _ICH)]],
                    rows[t].at[pl.ds(k * _ICH, _ICH), :],
                    sem))
        for c in copies:
            c.wait()
        for t in range(_NTAB):
            pltpu.sync_copy(rows[t], outs[t].at[pl.ds(base, _BPW), :])

    return gather_kernel(idx_flat, *tables)


_BLK = 1024
_H = 64


def _mlp_body(*refs):
    g_refs = refs[:_NTAB]
    time_ref, w1_ref, b1_ref, w2_ref, b2_ref, out_ref = refs[_NTAB:]
    acc = b1_ref[...]
    off = 0
    for t, d in enumerate(_DIMS):
        acc = acc + lax.dot_general(
            g_refs[t][...], w1_ref[off:off + d, :],
            (((1,), (0,)), ((), ())),
            precision=lax.Precision.HIGHEST,
            preferred_element_type=jnp.float32)
        off += d
    acc = acc + time_ref[...] * w1_ref[off:off + 1, :]
    h = jnp.maximum(acc, 0.0)
    o = lax.dot_general(
        h, w2_ref[...], (((1,), (0,)), ((), ())),
        precision=lax.Precision.HIGHEST,
        preferred_element_type=jnp.float32)
    out_ref[...] = jax.nn.sigmoid(o + b2_ref[...])


def _tc_mlp(gs, time_col, W1, b1, W2, b2):
    nblk = _B // _BLK
    in_specs = (
        [pl.BlockSpec((_BLK, d), lambda i: (i, 0)) for d in _DIMS]
        + [pl.BlockSpec((_BLK, 1), lambda i: (i, 0)),
           pl.BlockSpec((50, _H), lambda i: (0, 0)),
           pl.BlockSpec((1, _H), lambda i: (0, 0)),
           pl.BlockSpec((_H, 1), lambda i: (0, 0)),
           pl.BlockSpec((1, 1), lambda i: (0, 0))]
    )
    return pl.pallas_call(
        _mlp_body,
        grid=(nblk,),
        in_specs=in_specs,
        out_specs=pl.BlockSpec((_BLK, 1), lambda i: (i, 0)),
        out_shape=jax.ShapeDtypeStruct((_B, 1), jnp.float32),
    )(*gs, time_col, W1, b1, W2, b2)


def kernel(x, user_emb, product_emb, model_emb, gender_emb, age_emb,
           residence_emb, color_emb, size_emb, material_emb, W1, b1, W2, b2):
    idx_flat = x[:, :_NTAB].astype(jnp.int32).T.reshape(-1)
    tables = (user_emb, product_emb, model_emb, gender_emb, age_emb,
              residence_emb, color_emb, size_emb, material_emb)
    gs = _sc_gather(idx_flat, tables)
    time_col = x[:, _NTAB:_NTAB + 1]
    out = _tc_mlp(gs, time_col, W1, b1.reshape(1, _H), W2, b2.reshape(1, 1))
    return out[:, 0]


# speedup vs baseline: 2.0466x; 1.5878x over previous
import functools

import jax
import jax.numpy as jnp
from jax import lax
from jax.experimental import pallas as pl
from jax.experimental.pallas import tpu as pltpu
from jax.experimental.pallas import tpu_sc as plsc

_B = 16384
_NSC = 3
_D8 = 8
_NC, _NS = 2, 16
_NW = _NC * _NS
_BPW = _B // _NW
_ICH = 128
_NCHUNK = _BPW // _ICH


def _sc_gather(idx_flat, tables):
    mesh = plsc.VectorSubcoreMesh(
        core_axis_name="c", subcore_axis_name="s",
        num_cores=_NC, num_subcores=_NS)

    out_type = tuple(
        jax.ShapeDtypeStruct((_B, _D8), jnp.float32) for _ in range(_NSC))
    scratch = (
        [pltpu.VMEM((_NSC * _BPW,), jnp.int32)]
        + [pltpu.VMEM((_BPW, _D8), jnp.float32) for _ in range(_NSC)]
        + [pltpu.SemaphoreType.DMA, pltpu.SemaphoreType.DMA]
    )

    @functools.partial(
        pl.kernel, mesh=mesh, out_type=out_type, scratch_types=scratch,
        compiler_params=pltpu.CompilerParams(use_tc_tiling_on_sc=False))
    def gather_kernel(idx_hbm, *rest):
        tabs = rest[:_NSC]
        outs = rest[_NSC:2 * _NSC]
        idx_v = rest[2 * _NSC]
        rows = rest[2 * _NSC + 1:3 * _NSC + 1]
        sem = rest[3 * _NSC + 1]
        sem2 = rest[3 * _NSC + 2]

        wid = lax.axis_index("s") * _NC + lax.axis_index("c")
        base = wid * _BPW
        idx_copies = [
            pltpu.async_copy(
                idx_hbm.at[pl.ds(t * _B + base, _BPW)],
                idx_v.at[pl.ds(t * _BPW, _BPW)],
                sem2)
            for t in range(_NSC)
        ]
        for c in idx_copies:
            c.wait()
        copies = []
        for t in range(_NSC):
            for k in range(_NCHUNK):
                copies.append(pltpu.async_copy(
                    tabs[t].at[idx_v.at[pl.ds(t * _BPW + k * _ICH, _ICH)]],
                    rows[t].at[pl.ds(k * _ICH, _ICH), :],
                    sem))
        for c in copies:
            c.wait()
        for t in range(_NSC):
            pltpu.sync_copy(rows[t], outs[t].at[pl.ds(base, _BPW), :])

    return gather_kernel(idx_flat, *tables)


_BLK = 1024
_H = 64
_SMALL = ((2, 2, 24), (10, 4, 26), (5, 3, 30), (50, 8, 33), (20, 4, 41),
          (30, 4, 45))
_SMALL_XCOL = (3, 4, 5, 6, 7, 8)


def _mlp_body(*refs):
    g_refs = refs[:_NSC]
    x_ref = refs[_NSC]
    small_refs = refs[_NSC + 1:_NSC + 1 + len(_SMALL)]
    w1_ref, b1_ref, w2_ref, b2_ref, out_ref = refs[_NSC + 1 + len(_SMALL):]

    def dot(a, b):
        return lax.dot_general(
            a, b, (((1,), (0,)), ((), ())),
            precision=lax.Precision.HIGHEST,
            preferred_element_type=jnp.float32)

    acc = b1_ref[...]
    for t in range(_NSC):
        acc = acc + dot(g_refs[t][...], w1_ref[t * _D8:(t + 1) * _D8, :])
    for (v, d, off), c in zip(_SMALL, _SMALL_XCOL):
        ids = x_ref[:, c:c + 1].astype(jnp.int32)
        onehot = jnp.where(
            ids == lax.broadcasted_iota(jnp.int32, (1, v), 1), 1.0, 0.0)
        acc = acc + dot(dot(onehot, small_refs[_SMALL.index((v, d, off))][...]),
                        w1_ref[off:off + d, :])
    acc = acc + x_ref[:, 9:10] * w1_ref[49:50, :]
    h = jnp.maximum(acc, 0.0)
    o = dot(h, w2_ref[...])
    out_ref[...] = jax.nn.sigmoid(o + b2_ref[...])


def _tc_mlp(gs, x, smalls, W1, b1, W2, b2):
    nblk = _B // _BLK
    in_specs = (
        [pl.BlockSpec((_BLK, _D8), lambda i: (i, 0)) for _ in range(_NSC)]
        + [pl.BlockSpec((_BLK, 10), lambda i: (i, 0))]
        + [pl.BlockSpec((v, d), lambda i: (0, 0)) for (v, d, _) in _SMALL]
        + [pl.BlockSpec((50, _H), lambda i: (0, 0)),
           pl.BlockSpec((1, _H), lambda i: (0, 0)),
           pl.BlockSpec((_H, 1), lambda i: (0, 0)),
           pl.BlockSpec((1, 1), lambda i: (0, 0))]
    )
    return pl.pallas_call(
        _mlp_body,
        grid=(nblk,),
        in_specs=in_specs,
        out_specs=pl.BlockSpec((_BLK, 1), lambda i: (i, 0)),
        out_shape=jax.ShapeDtypeStruct((_B, 1), jnp.float32),
    )(*gs, x, *smalls, W1, b1, W2, b2)


def kernel(x, user_emb, product_emb, model_emb, gender_emb, age_emb,
           residence_emb, color_emb, size_emb, material_emb, W1, b1, W2, b2):
    idx_flat = x[:, :_NSC].astype(jnp.int32).T.reshape(-1)
    gs = _sc_gather(idx_flat, (user_emb, product_emb, model_emb))
    smalls = (gender_emb, age_emb, residence_emb, color_emb, size_emb,
              material_emb)
    out = _tc_mlp(gs, x, smalls, W1, b1.reshape(1, _H), W2, b2.reshape(1, 1))
    return out[:, 0]


# speedup vs baseline: 3.1946x; 1.5609x over previous
import functools

import jax
import jax.numpy as jnp
from jax import lax
from jax.experimental import pallas as pl
from jax.experimental.pallas import tpu as pltpu
from jax.experimental.pallas import tpu_sc as plsc

_B = 16384
_NC, _NS = 2, 16
_NW = _NC * _NS
_BPW = _B // _NW
_ICH = 128
_NCHUNK = _BPW // _ICH
_NBIG = 3
_D8 = 8
_F = 50
_NGRP = _BPW // 16

_SMALL = (
    (2, 2, 0, 24, 3),
    (10, 4, 8, 26, 4),
    (5, 3, 48, 30, 5),
    (50, 8, 64, 33, 6),
    (20, 4, 464, 41, 7),
    (30, 4, 544, 45, 8),
)
_SMALLCAT_WORDS = 664


def _sc_embed(x, smallcat, big_tables):
    mesh = plsc.VectorSubcoreMesh(
        core_axis_name="c", subcore_axis_name="s",
        num_cores=_NC, num_subcores=_NS)

    scratch = (
        [pltpu.VMEM((_BPW, 10), jnp.float32),
         pltpu.VMEM((_SMALLCAT_WORDS,), jnp.float32),
         pltpu.VMEM((_NBIG * _BPW,), jnp.int32)]
        + [pltpu.VMEM((_BPW, _D8), jnp.float32) for _ in range(_NBIG)]
        + [pltpu.VMEM((_F, _BPW), jnp.float32),
           pltpu.SemaphoreType.DMA, pltpu.SemaphoreType.DMA]
    )

    @functools.partial(
        pl.kernel, mesh=mesh,
        out_type=jax.ShapeDtypeStruct((_F, _B), jnp.float32),
        scratch_types=scratch,
        compiler_params=pltpu.CompilerParams(
            use_tc_tiling_on_sc=False, needs_layout_passes=False))
    def embed_kernel(x_hbm, small_hbm, t0, t1, t2, out_hbm,
                     x_v, tab_v, idx_v, r0, r1, r2, feats_v, sem, sem2):
        tabs = (t0, t1, t2)
        rows = (r0, r1, r2)
        wid = lax.axis_index("s") * _NC + lax.axis_index("c")
        base = wid * _BPW

        cp_x = pltpu.async_copy(x_hbm.at[pl.ds(base, _BPW), :], x_v, sem2)
        cp_s = pltpu.async_copy(small_hbm, tab_v, sem2)
        cp_x.wait()
        cp_s.wait()

        iota = lax.iota(jnp.int32, 16)

        def splat(v):
            return jnp.full((16,), v, jnp.int32)

        def extract(g, _):
            rowv = splat(g * 16) + iota
            for t in range(_NBIG):
                f = plsc.load_gather(x_v, [rowv, splat(t)])
                plsc.store_scatter(idx_v, [rowv + splat(t * _BPW)],
                                   f.astype(jnp.int32))
            for (v, d, woff, foff, xcol) in _SMALL:
                f = plsc.load_gather(x_v, [rowv, splat(xcol)])
                wbase = f.astype(jnp.int32) * d + splat(woff)
                for k in range(d):
                    val = plsc.load_gather(tab_v, [wbase + splat(k)])
                    plsc.store_scatter(feats_v, [splat(foff + k), rowv], val)
            tv = plsc.load_gather(x_v, [rowv, splat(9)])
            plsc.store_scatter(feats_v, [splat(_F - 1), rowv], tv)
            return _

        lax.fori_loop(0, _NGRP, extract, None)

        copies = []
        for t in range(_NBIG):
            for k in range(_NCHUNK):
                copies.append(pltpu.async_copy(
                    tabs[t].at[idx_v.at[pl.ds(t * _BPW + k * _ICH, _ICH)]],
                    rows[t].at[pl.ds(k * _ICH, _ICH), :],
                    sem))
        for c in copies:
            c.wait()

        def xpose(g, _):
            rowv = splat(g * 16) + iota
            for t in range(_NBIG):
                for k in range(_D8):
                    val = plsc.load_gather(rows[t], [rowv, splat(k)])
                    plsc.store_scatter(
                        feats_v, [splat(t * _D8 + k), rowv], val)
            return _

        lax.fori_loop(0, _NGRP, xpose, None)

        pltpu.sync_copy(feats_v, out_hbm.at[:, pl.ds(base, _BPW)])

    return embed_kernel(x, smallcat, *big_tables)


_BLK = 2048


def _mlp_body(f_ref, w1t_ref, b1_ref, w2t_ref, b2_ref, out_ref):
    def dot(a, b):
        return lax.dot_general(
            a, b, (((1,), (0,)), ((), ())),
            precision=lax.Precision.HIGHEST,
            preferred_element_type=jnp.float32)

    h = jnp.maximum(dot(w1t_ref[...], f_ref[...]) + b1_ref[...], 0.0)
    out_ref[...] = jax.nn.sigmoid(dot(w2t_ref[...], h) + b2_ref[...])


def _tc_mlp(featsT, W1T, b1c, W2T, b2c):
    nblk = _B // _BLK
    return pl.pallas_call(
        _mlp_body,
        grid=(nblk,),
        in_specs=[
            pl.BlockSpec((_F, _BLK), lambda i: (0, i)),
            pl.BlockSpec((64, _F), lambda i: (0, 0)),
            pl.BlockSpec((64, 1), lambda i: (0, 0)),
            pl.BlockSpec((1, 64), lambda i: (0, 0)),
            pl.BlockSpec((1, 1), lambda i: (0, 0)),
        ],
        out_specs=pl.BlockSpec((1, _BLK), lambda i: (0, i)),
        out_shape=jax.ShapeDtypeStruct((1, _B), jnp.float32),
    )(featsT, W1T, b1c, W2T, b2c)


def _pad_to(a, n):
    return jnp.concatenate([a, jnp.zeros((n - a.shape[0],), a.dtype)])


def kernel(x, user_emb, product_emb, model_emb, gender_emb, age_emb,
           residence_emb, color_emb, size_emb, material_emb, W1, b1, W2, b2):
    smallcat = jnp.concatenate([
        _pad_to(gender_emb.reshape(-1), 8),
        _pad_to(age_emb.reshape(-1), 40),
        _pad_to(residence_emb.reshape(-1), 16),
        color_emb.reshape(-1),
        size_emb.reshape(-1),
        material_emb.reshape(-1),
    ])
    featsT = _sc_embed(x, smallcat, (user_emb, product_emb, model_emb))
    out = _tc_mlp(featsT, W1.T, b1.reshape(64, 1), W2.T, b2.reshape(1, 1))
    return out[0]


# speedup vs baseline: 7.3240x; 2.2926x over previous
import functools

import jax
import jax.numpy as jnp
from jax import lax
from jax.experimental import pallas as pl
from jax.experimental.pallas import tpu as pltpu
from jax.experimental.pallas import tpu_sc as plsc

_B = 16384
_NC, _NS = 2, 16
_NW = _NC * _NS
_BPW = _B // _NW
_ICH = 128
_NCHUNK = _BPW // _ICH
_NBIG = 3
_D8 = 8
_BIGV = (100000, 100000, 1000)
_F = 50
_NGRP = _BPW // 16

_SMALL = (
    (2, 2, 0, 24, 3),
    (10, 4, 8, 26, 4),
    (5, 3, 48, 30, 5),
    (50, 8, 64, 33, 6),
    (20, 4, 464, 41, 7),
    (30, 4, 544, 45, 8),
)
_SMALLCAT_WORDS = 664


def _sc_embed(xT, smallcat, big_flats):
    mesh = plsc.VectorSubcoreMesh(
        core_axis_name="c", subcore_axis_name="s",
        num_cores=_NC, num_subcores=_NS)

    scratch = (
        [pltpu.VMEM((10, _BPW), jnp.float32),
         pltpu.VMEM((_SMALLCAT_WORDS,), jnp.float32),
         pltpu.VMEM((_NBIG * _D8 * _BPW,), jnp.int32),
         pltpu.VMEM((_F, _BPW), jnp.float32),
         pltpu.SemaphoreType.DMA, pltpu.SemaphoreType.DMA]
    )

    @functools.partial(
        pl.kernel, mesh=mesh,
        out_type=jax.ShapeDtypeStruct((_F, _B), jnp.float32),
        scratch_types=scratch,
        compiler_params=pltpu.CompilerParams(
            use_tc_tiling_on_sc=False, needs_layout_passes=False))
    def embed_kernel(x_hbm, small_hbm, t0, t1, t2, out_hbm,
                     x_v, tab_v, idx_v, feats_v, sem, sem2):
        tabs = (t0, t1, t2)
        wid = lax.axis_index("s") * _NC + lax.axis_index("c")
        base = wid * _BPW

        cp_x = pltpu.async_copy(x_hbm.at[:, pl.ds(base, _BPW)], x_v, sem2)
        cp_s = pltpu.async_copy(small_hbm, tab_v, sem2)
        cp_x.wait()
        cp_s.wait()

        def splat(v):
            return jnp.full((16,), v, jnp.int32)

        def extract(g, carry):
            o16 = g * 16
            for t in range(_NBIG):
                idx = x_v[t, pl.ds(o16, 16)].astype(jnp.int32)
                for d in range(_D8):
                    idx_v[pl.ds((t * _D8 + d) * _BPW + o16, 16)] = (
                        idx + splat(d * _BIGV[t]))
            for (v, d, woff, foff, xrow) in _SMALL:
                idx = x_v[xrow, pl.ds(o16, 16)].astype(jnp.int32)
                for k in range(d):
                    val = plsc.load_gather(
                        tab_v, [idx + splat(woff + k * v)])
                    feats_v[foff + k, pl.ds(o16, 16)] = val
            feats_v[_F - 1, pl.ds(o16, 16)] = x_v[9, pl.ds(o16, 16)]
            return carry

        lax.fori_loop(0, _NGRP, extract, None)

        copies = []
        for t in range(_NBIG):
            for d in range(_D8):
                for k in range(_NCHUNK):
                    copies.append(pltpu.async_copy(
                        tabs[t].at[idx_v.at[pl.ds(
                            (t * _D8 + d) * _BPW + k * _ICH, _ICH)]],
                        feats_v.at[t * _D8 + d, pl.ds(k * _ICH, _ICH)],
                        sem))
        for c in copies:
            c.wait()

        pltpu.sync_copy(feats_v, out_hbm.at[:, pl.ds(base, _BPW)])

    return embed_kernel(xT, smallcat, *big_flats)


_BLK = 2048


def _mlp_body(f_ref, w1t_ref, b1_ref, w2t_ref, b2_ref, out_ref):
    def dot(a, b):
        return lax.dot_general(
            a, b, (((1,), (0,)), ((), ())),
            precision=lax.Precision.HIGHEST,
            preferred_element_type=jnp.float32)

    h = jnp.maximum(dot(w1t_ref[...], f_ref[...]) + b1_ref[...], 0.0)
    out_ref[...] = jax.nn.sigmoid(dot(w2t_ref[...], h) + b2_ref[...])


def _tc_mlp(featsT, W1T, b1c, W2T, b2c):
    nblk = _B // _BLK
    return pl.pallas_call(
        _mlp_body,
        grid=(nblk,),
        in_specs=[
            pl.BlockSpec((_F, _BLK), lambda i: (0, i)),
            pl.BlockSpec((64, _F), lambda i: (0, 0)),
            pl.BlockSpec((64, 1), lambda i: (0, 0)),
            pl.BlockSpec((1, 64), lambda i: (0, 0)),
            pl.BlockSpec((1, 1), lambda i: (0, 0)),
        ],
        out_specs=pl.BlockSpec((1, _BLK), lambda i: (0, i)),
        out_shape=jax.ShapeDtypeStruct((1, _B), jnp.float32),
    )(featsT, W1T, b1c, W2T, b2c)


def _pad_to(a, n):
    return jnp.concatenate([a, jnp.zeros((n - a.shape[0],), a.dtype)])


def kernel(x, user_emb, product_emb, model_emb, gender_emb, age_emb,
           residence_emb, color_emb, size_emb, material_emb, W1, b1, W2, b2):
    smallcat = jnp.concatenate([
        _pad_to(gender_emb.T.reshape(-1), 8),
        _pad_to(age_emb.T.reshape(-1), 40),
        _pad_to(residence_emb.T.reshape(-1), 16),
        color_emb.T.reshape(-1),
        size_emb.T.reshape(-1),
        material_emb.T.reshape(-1),
    ])
    featsT = _sc_embed(
        x.T, smallcat,
        (user_emb.T.reshape(-1), product_emb.T.reshape(-1),
         model_emb.T.reshape(-1)))
    out = _tc_mlp(featsT, W1.T, b1.reshape(64, 1), W2.T, b2.reshape(1, 1))
    return out[0]


# speedup vs baseline: 10.9496x; 1.4950x over previous
import functools

import jax
import jax.numpy as jnp
from jax import lax
from jax.experimental import pallas as pl
from jax.experimental.pallas import tpu as pltpu
from jax.experimental.pallas import tpu_sc as plsc

_B = 16384
_NC, _NS = 2, 16
_NW = _NC * _NS
_BPW = _B // _NW
_ICH = 128
_NCHUNK = _BPW // _ICH
_NBIG = 2
_D8 = 8
_BIGV = (100000, 100000)
_F = 50
_NGRP = _BPW // 16

_SMALL = (
    (1000, 8, 0, 16, 2),
    (2, 2, 8000, 24, 3),
    (10, 4, 8008, 26, 4),
    (5, 3, 8048, 30, 5),
    (50, 8, 8064, 33, 6),
    (20, 4, 8464, 41, 7),
    (30, 4, 8544, 45, 8),
)
_SMALLCAT_WORDS = 8664


def _sc_embed(xT, smallcat, big_flats):
    mesh = plsc.VectorSubcoreMesh(
        core_axis_name="c", subcore_axis_name="s",
        num_cores=_NC, num_subcores=_NS)

    scratch = (
        [pltpu.VMEM((10, _BPW), jnp.float32),
         pltpu.VMEM((_SMALLCAT_WORDS,), jnp.float32),
         pltpu.VMEM((_NBIG * _D8 * _BPW,), jnp.int32),
         pltpu.VMEM((_F, _BPW), jnp.float32),
         pltpu.SemaphoreType.DMA, pltpu.SemaphoreType.DMA]
    )

    @functools.partial(
        pl.kernel, mesh=mesh,
        out_type=jax.ShapeDtypeStruct((_F, _B), jnp.float32),
        scratch_types=scratch,
        compiler_params=pltpu.CompilerParams(
            use_tc_tiling_on_sc=False, needs_layout_passes=False))
    def embed_kernel(x_hbm, small_hbm, t0, t1, out_hbm,
                     x_v, tab_v, idx_v, feats_v, sem, sem2):
        tabs = (t0, t1)
        wid = lax.axis_index("s") * _NC + lax.axis_index("c")
        base = wid * _BPW

        cp_x = pltpu.async_copy(x_hbm.at[:, pl.ds(base, _BPW)], x_v, sem2)
        cp_s = pltpu.async_copy(small_hbm, tab_v, sem2)
        cp_x.wait()

        def splat(v):
            return jnp.full((16,), v, jnp.int32)

        def extract_big(g, carry):
            o16 = g * 16
            for t in range(_NBIG):
                idx = x_v[t, pl.ds(o16, 16)].astype(jnp.int32)
                for d in range(_D8):
                    idx_v[pl.ds((t * _D8 + d) * _BPW + o16, 16)] = (
                        idx + splat(d * _BIGV[t]))
            return carry

        lax.fori_loop(0, _NGRP, extract_big, None)

        copies = []
        for t in range(_NBIG):
            for d in range(_D8):
                for k in range(_NCHUNK):
                    copies.append(pltpu.async_copy(
                        tabs[t].at[idx_v.at[pl.ds(
                            (t * _D8 + d) * _BPW + k * _ICH, _ICH)]],
                        feats_v.at[t * _D8 + d, pl.ds(k * _ICH, _ICH)],
                        sem))

        cp_s.wait()

        def extract_small(g, carry):
            o16 = g * 16
            for (v, d, woff, foff, xrow) in _SMALL:
                idx = x_v[xrow, pl.ds(o16, 16)].astype(jnp.int32)
                for k in range(d):
                    val = plsc.load_gather(
                        tab_v, [idx + splat(woff + k * v)])
                    feats_v[foff + k, pl.ds(o16, 16)] = val
            feats_v[_F - 1, pl.ds(o16, 16)] = x_v[9, pl.ds(o16, 16)]
            return carry

        lax.fori_loop(0, _NGRP, extract_small, None)

        for c in copies:
            c.wait()

        pltpu.sync_copy(feats_v, out_hbm.at[:, pl.ds(base, _BPW)])

    return embed_kernel(xT, smallcat, *big_flats)


_BLK = 4096


def _mlp_body(f_ref, w1t_ref, b1_ref, w2t_ref, b2_ref, out_ref):
    def dot(a, b):
        return lax.dot_general(
            a, b, (((1,), (0,)), ((), ())),
            precision=lax.Precision.HIGHEST,
            preferred_element_type=jnp.float32)

    h = jnp.maximum(dot(w1t_ref[...], f_ref[...]) + b1_ref[...], 0.0)
    out_ref[...] = jax.nn.sigmoid(dot(w2t_ref[...], h) + b2_ref[...])


def _tc_mlp(featsT, W1T, b1c, W2T, b2c):
    nblk = _B // _BLK
    return pl.pallas_call(
        _mlp_body,
        grid=(nblk,),
        in_specs=[
            pl.BlockSpec((_F, _BLK), lambda i: (0, i)),
            pl.BlockSpec((64, _F), lambda i: (0, 0)),
            pl.BlockSpec((64, 1), lambda i: (0, 0)),
            pl.BlockSpec((1, 64), lambda i: (0, 0)),
            pl.BlockSpec((1, 1), lambda i: (0, 0)),
        ],
        out_specs=pl.BlockSpec((1, _BLK), lambda i: (0, i)),
        out_shape=jax.ShapeDtypeStruct((1, _B), jnp.float32),
    )(featsT, W1T, b1c, W2T, b2c)


def _pad_to(a, n):
    return jnp.concatenate([a, jnp.zeros((n - a.shape[0],), a.dtype)])


def kernel(x, user_emb, product_emb, model_emb, gender_emb, age_emb,
           residence_emb, color_emb, size_emb, material_emb, W1, b1, W2, b2):
    smallcat = jnp.concatenate([
        model_emb.T.reshape(-1),
        _pad_to(gender_emb.T.reshape(-1), 8),
        _pad_to(age_emb.T.reshape(-1), 40),
        _pad_to(residence_emb.T.reshape(-1), 16),
        color_emb.T.reshape(-1),
        size_emb.T.reshape(-1),
        material_emb.T.reshape(-1),
    ])
    featsT = _sc_embed(
        x.T, smallcat,
        (user_emb.T.reshape(-1), product_emb.T.reshape(-1)))
    out = _tc_mlp(featsT, W1.T, b1.reshape(64, 1), W2.T, b2.reshape(1, 1))
    return out[0]


# speedup vs baseline: 10.9770x; 1.0025x over previous
import functools

import jax
import jax.numpy as jnp
from jax import lax
from jax.experimental import pallas as pl
from jax.experimental.pallas import tpu as pltpu
from jax.experimental.pallas import tpu_sc as plsc

_B = 16384
_NC, _NS = 2, 16
_NW = _NC * _NS
_BPW = _B // _NW
_ICH = 128
_NCHUNK = _BPW // _ICH
_NBIG = 2
_D8 = 8
_BIGV = (100000, 100000)
_F = 50
_NGRP = _BPW // 16

_SMALL = (
    (1000, 8, 0, 16, 2),
    (2, 2, 8000, 24, 3),
    (10, 4, 8008, 26, 4),
    (5, 3, 8048, 30, 5),
    (50, 8, 8064, 33, 6),
    (20, 4, 8464, 41, 7),
    (30, 4, 8544, 45, 8),
)
_SMALLCAT_WORDS = 8664


def _sc_embed(xT, smallcat, big_flats):
    mesh = plsc.VectorSubcoreMesh(
        core_axis_name="c", subcore_axis_name="s",
        num_cores=_NC, num_subcores=_NS)

    scratch = (
        [pltpu.VMEM((10, _BPW), jnp.float32),
         pltpu.VMEM((_SMALLCAT_WORDS,), jnp.float32),
         pltpu.VMEM((_NBIG * _D8 * _BPW,), jnp.int32),
         pltpu.VMEM((_F, _BPW), jnp.float32),
         pltpu.SemaphoreType.DMA, pltpu.SemaphoreType.DMA]
    )

    @functools.partial(
        pl.kernel, mesh=mesh,
        out_type=jax.ShapeDtypeStruct((_F, _B), jnp.float32),
        scratch_types=scratch,
        compiler_params=pltpu.CompilerParams(
            use_tc_tiling_on_sc=False, needs_layout_passes=False))
    def embed_kernel(x_hbm, small_hbm, t0, t1, out_hbm,
                     x_v, tab_v, idx_v, feats_v, sem, sem2):
        tabs = (t0, t1)
        wid = lax.axis_index("s") * _NC + lax.axis_index("c")
        base = wid * _BPW

        cp_x = pltpu.async_copy(x_hbm.at[:, pl.ds(base, _BPW)], x_v, sem2)
        cp_s = pltpu.async_copy(small_hbm, tab_v, sem2)
        cp_x.wait()

        def splat(v):
            return jnp.full((16,), v, jnp.int32)

        copies = []
        for k in range(_NCHUNK):
            def extract_chunk(g, carry, k=k):
                o16 = k * _ICH + g * 16
                for t in range(_NBIG):
                    idx = x_v[t, pl.ds(o16, 16)].astype(jnp.int32)
                    for d in range(_D8):
                        idx_v[pl.ds((t * _D8 + d) * _BPW + o16, 16)] = (
                            idx + splat(d * _BIGV[t]))
                return carry

            lax.fori_loop(0, _ICH // 16, extract_chunk, None)
            for t in range(_NBIG):
                for d in range(_D8):
                    copies.append(pltpu.async_copy(
                        tabs[t].at[idx_v.at[pl.ds(
                            (t * _D8 + d) * _BPW + k * _ICH, _ICH)]],
                        feats_v.at[t * _D8 + d, pl.ds(k * _ICH, _ICH)],
                        sem))

        cp_s.wait()

        def extract_small(g, carry):
            o16 = g * 16
            for (v, d, woff, foff, xrow) in _SMALL:
                idx = x_v[xrow, pl.ds(o16, 16)].astype(jnp.int32)
                for k in range(d):
                    val = plsc.load_gather(
                        tab_v, [idx + splat(woff + k * v)])
                    feats_v[foff + k, pl.ds(o16, 16)] = val
            feats_v[_F - 1, pl.ds(o16, 16)] = x_v[9, pl.ds(o16, 16)]
            return carry

        lax.fori_loop(0, _NGRP, extract_small, None)

        nbr = _NBIG * _D8
        cp_out = pltpu.async_copy(
            feats_v.at[pl.ds(nbr, _F - nbr)],
            out_hbm.at[pl.ds(nbr, _F - nbr), pl.ds(base, _BPW)], sem2)

        for c in copies:
            c.wait()

        pltpu.sync_copy(feats_v.at[pl.ds(0, nbr)],
                        out_hbm.at[pl.ds(0, nbr), pl.ds(base, _BPW)])
        cp_out.wait()

    return embed_kernel(xT, smallcat, *big_flats)


_BLK = 4096


def _mlp_body(f_ref, w1t_ref, b1_ref, w2t_ref, b2_ref, out_ref):
    def dot(a, b):
        return lax.dot_general(
            a, b, (((1,), (0,)), ((), ())),
            precision=lax.Precision.HIGHEST,
            preferred_element_type=jnp.float32)

    h = jnp.maximum(dot(w1t_ref[...], f_ref[...]) + b1_ref[...], 0.0)
    out_ref[...] = jax.nn.sigmoid(dot(w2t_ref[...], h) + b2_ref[...])


def _tc_mlp(featsT, W1T, b1c, W2T, b2c):
    nblk = _B // _BLK
    return pl.pallas_call(
        _mlp_body,
        grid=(nblk,),
        in_specs=[
            pl.BlockSpec((_F, _BLK), lambda i: (0, i)),
            pl.BlockSpec((64, _F), lambda i: (0, 0)),
            pl.BlockSpec((64, 1), lambda i: (0, 0)),
            pl.BlockSpec((1, 64), lambda i: (0, 0)),
            pl.BlockSpec((1, 1), lambda i: (0, 0)),
        ],
        out_specs=pl.BlockSpec((1, _BLK), lambda i: (0, i)),
        out_shape=jax.ShapeDtypeStruct((1, _B), jnp.float32),
    )(featsT, W1T, b1c, W2T, b2c)


def _pad_to(a, n):
    return jnp.concatenate([a, jnp.zeros((n - a.shape[0],), a.dtype)])


def kernel(x, user_emb, product_emb, model_emb, gender_emb, age_emb,
           residence_emb, color_emb, size_emb, material_emb, W1, b1, W2, b2):
    smallcat = jnp.concatenate([
        model_emb.T.reshape(-1),
        _pad_to(gender_emb.T.reshape(-1), 8),
        _pad_to(age_emb.T.reshape(-1), 40),
        _pad_to(residence_emb.T.reshape(-1), 16),
        color_emb.T.reshape(-1),
        size_emb.T.reshape(-1),
        material_emb.T.reshape(-1),
    ])
    featsT = _sc_embed(
        x.T, smallcat,
        (user_emb.T.reshape(-1), product_emb.T.reshape(-1)))
    out = _tc_mlp(featsT, W1.T, b1.reshape(64, 1), W2.T, b2.reshape(1, 1))
    return out[0]
